# baseline (speedup 1.0000x reference)
import functools

import jax
import jax.numpy as jnp
from jax import lax
from jax.experimental import pallas as pl
from jax.experimental.pallas import tpu as pltpu
from jax.experimental.pallas import tpu_sc as plsc

NUM_DOMAINS = 26
BATCH = 16384
PAD = 128
NC, NS, L = 2, 16, 16
NW = NC * NS
CHUNK = BATCH // NW
STEPS = CHUNK // L


def _table_body(w_ref, out_ref):
    w = w_ref[...]
    sp = jnp.maximum(w, 0.0) + jnp.log1p(jnp.exp(-jnp.abs(w)))
    mask = lax.broadcasted_iota(jnp.int32, (1, PAD), 1) < NUM_DOMAINS
    total = jnp.sum(jnp.where(mask, sp, 0.0))
    out_ref[...] = sp * (NUM_DOMAINS / total)


def _make_table(w_padded):
    return pl.pallas_call(
        _table_body,
        out_shape=jax.ShapeDtypeStruct((1, PAD), jnp.float32),
    )(w_padded)


_sc_mesh = plsc.VectorSubcoreMesh(
    core_axis_name="c", subcore_axis_name="s", num_cores=NC, num_subcores=NS
)


@functools.partial(
    pl.kernel,
    out_type=jax.ShapeDtypeStruct((BATCH,), jnp.float32),
    mesh=_sc_mesh,
    scratch_types=[
        pltpu.VMEM((CHUNK,), jnp.int32),
        pltpu.VMEM((CHUNK,), jnp.float32),
        pltpu.VMEM((CHUNK,), jnp.float32),
        pltpu.VMEM((PAD,), jnp.float32),
    ],
    compiler_params=pltpu.CompilerParams(needs_layout_passes=False),
)
def _sc_gather_mul(idx_hbm, x_hbm, tab_hbm, out_hbm, idx_v, x_v, out_v, tab_v):
    wid = lax.axis_index("s") * NC + lax.axis_index("c")
    base = wid * CHUNK
    pltpu.sync_copy(tab_hbm, tab_v)
    pltpu.sync_copy(idx_hbm.at[pl.ds(base, CHUNK)], idx_v)
    pltpu.sync_copy(x_hbm.at[pl.ds(base, CHUNK)], x_v)
    for i in range(STEPS):
        sl = pl.ds(i * L, L)
        w = plsc.load_gather(tab_v, [idx_v[sl]])
        out_v[sl] = x_v[sl] * w
    pltpu.sync_copy(out_v, out_hbm.at[pl.ds(base, CHUNK)])


def kernel(idxes, x, raw_weights):
    w_padded = jnp.pad(raw_weights, (0, PAD - NUM_DOMAINS)).reshape(1, PAD)
    table = _make_table(w_padded).reshape(PAD)
    out = _sc_gather_mul(idxes, x.reshape(BATCH), table)
    return out.reshape(BATCH, 1)


# speedup vs baseline: 1.0624x; 1.0624x over previous
import functools

import jax
import jax.numpy as jnp
from jax import lax
from jax.experimental import pallas as pl
from jax.experimental.pallas import tpu as pltpu
from jax.experimental.pallas import tpu_sc as plsc

NUM_DOMAINS = 26
BATCH = 16384
NC, NS, L = 2, 16, 16
NW = NC * NS
CHUNK = BATCH // NW
STEPS = CHUNK // L


def _log1p_unit(u):
    z = 1.0 + u
    y = u * (1.0 - u * (0.5 - u * (1.0 / 3.0)))
    for _ in range(3):
        y = y - 1.0 + z * jnp.exp(-y)
    return y


def _softplus(w):
    return jnp.maximum(w, 0.0) + _log1p_unit(jnp.exp(-jnp.abs(w)))


_sc_mesh = plsc.VectorSubcoreMesh(
    core_axis_name="c", subcore_axis_name="s", num_cores=NC, num_subcores=NS
)


@functools.partial(
    pl.kernel,
    out_type=jax.ShapeDtypeStruct((BATCH,), jnp.float32),
    mesh=_sc_mesh,
    scratch_types=[
        pltpu.VMEM((CHUNK,), jnp.int32),
        pltpu.VMEM((CHUNK,), jnp.float32),
        pltpu.VMEM((CHUNK,), jnp.float32),
        pltpu.VMEM((NUM_DOMAINS,), jnp.float32),
        pltpu.VMEM((2 * L,), jnp.float32),
        pltpu.SemaphoreType.DMA,
        pltpu.SemaphoreType.DMA,
        pltpu.SemaphoreType.DMA,
    ],
    compiler_params=pltpu.CompilerParams(needs_layout_passes=False),
)
def _sc_kernel(idx_hbm, x_hbm, raw_hbm, out_hbm,
               idx_v, x_v, out_v, raw_v, tab_v, sem0, sem1, sem2):
    wid = lax.axis_index("s") * NC + lax.axis_index("c")
    base = wid * CHUNK
    cp_raw = pltpu.async_copy(raw_hbm, raw_v, sem0)
    cp_idx = pltpu.async_copy(idx_hbm.at[pl.ds(base, CHUNK)], idx_v, sem1)
    cp_x = pltpu.async_copy(x_hbm.at[pl.ds(base, CHUNK)], x_v, sem2)
    cp_raw.wait()

    lane = lax.broadcasted_iota(jnp.int32, (L,), 0)
    idx_hi = jnp.minimum(lane + L, NUM_DOMAINS - 1)
    w_lo = plsc.load_gather(raw_v, [lane])
    w_hi = plsc.load_gather(raw_v, [idx_hi])
    mask_hi = (lane + L) < NUM_DOMAINS
    sp_lo = _softplus(w_lo)
    sp_hi = jnp.where(mask_hi, _softplus(w_hi), 0.0)
    total = jnp.broadcast_to(jnp.sum(sp_lo) + jnp.sum(sp_hi), (L,))
    scale = NUM_DOMAINS / total
    tab_v[pl.ds(0, L)] = sp_lo * scale
    tab_v[pl.ds(L, L)] = sp_hi * scale

    cp_idx.wait()
    cp_x.wait()
    for i in range(STEPS):
        sl = pl.ds(i * L, L)
        w = plsc.load_gather(tab_v, [idx_v[sl]])
        out_v[sl] = x_v[sl] * w
    pltpu.sync_copy(out_v, out_hbm.at[pl.ds(base, CHUNK)])


def kernel(idxes, x, raw_weights):
    out = _sc_kernel(idxes, x.reshape(BATCH), raw_weights)
    return out.reshape(BATCH, 1)
